# baseline (speedup 1.0000x reference)
import functools

import jax
import jax.numpy as jnp
from jax import lax
from jax.experimental import pallas as pl
from jax.experimental.pallas import tpu as pltpu
from jax.experimental.pallas import tpu_sc as plsc

_NUM_CORES = 2
_NUM_SUBCORES = 16
_NW = _NUM_CORES * _NUM_SUBCORES
_CHUNK = 16
_GSUB = 1
_NBUF = 14


@functools.partial(jax.jit, static_argnames=())
def _sc_gather(idx_flat, codebook):
    B = idx_flat.shape[0]
    V, D = codebook.shape
    b_per_w = B // _NW
    n_chunks = b_per_w // _CHUNK
    C = _CHUNK
    mesh = plsc.VectorSubcoreMesh(core_axis_name="c", subcore_axis_name="s")

    nb = _NBUF
    ns = _GSUB
    S = C // ns
    row_bufs = [pltpu.VMEM((C, D), jnp.float32) for _ in range(nb)]
    g_sems = [pltpu.SemaphoreType.DMA for _ in range(nb)]
    w_sems = [pltpu.SemaphoreType.DMA for _ in range(nb)]

    @functools.partial(
        pl.kernel,
        out_type=jax.ShapeDtypeStruct((B, D), jnp.float32),
        mesh=mesh,
        scratch_types=[pltpu.VMEM((b_per_w,), jnp.int32)] + row_bufs + g_sems + w_sems,
    )
    def k(table_hbm, idx_hbm, out_hbm, idx_v, *bufs_and_sems):
        rows = bufs_and_sems[:nb]
        sg = bufs_and_sems[nb : 2 * nb]
        sw = bufs_and_sems[2 * nb : 3 * nb]
        wid = lax.axis_index("s") * _NUM_CORES + lax.axis_index("c")
        base = wid * b_per_w
        pltpu.sync_copy(idx_hbm.at[pl.ds(base, b_per_w)], idx_v)

        def gather(off, b):
            for h in range(ns):
                pltpu.async_copy(
                    table_hbm.at[idx_v.at[pl.ds(off + h * S, S)]],
                    rows[b].at[pl.ds(h * S, S)],
                    sg[b],
                )

        def wait_gather(b):
            for h in range(ns):
                pltpu.make_async_copy(
                    table_hbm.at[idx_v.at[pl.ds(0, S)]],
                    rows[b].at[pl.ds(0, S)],
                    sg[b],
                ).wait()

        def write(off, b):
            pltpu.async_copy(rows[b], out_hbm.at[pl.ds(base + off, C)], sw[b])

        def wait_write(b):
            pltpu.make_async_copy(rows[b], out_hbm.at[pl.ds(base, C)], sw[b]).wait()

        for b in range(nb):
            gather(b * C, b)

        n_main = (n_chunks // nb) - 1
        tail = n_chunks - (n_main + 1) * nb

        lag = 5
        @pl.loop(0, n_main)
        def _(g):
            off = g * nb * C
            for b in range(nb):
                wait_gather(b)
                write(off + b * C, b)
                bb = b - lag
                if bb >= 0:
                    wait_write(bb)
                    gather(off + (nb + bb) * C, bb)
            for bb in range(nb - lag, nb):
                wait_write(bb)
                gather(off + (nb + bb) * C, bb)

        last = n_main * nb * C
        for b in range(nb):
            wait_gather(b)
            write(last + b * C, b)
        for b in range(tail):
            wait_write(b)
            gather(last + (nb + b) * C, b)
        for b in range(tail):
            wait_gather(b)
            write(last + (nb + b) * C, b)
        for b in range(nb):
            wait_write(b)

    return k(codebook, idx_flat)


def kernel(indices, codebook):
    shape = indices.shape
    idx_flat = indices.reshape(-1).astype(jnp.int32)
    out = _sc_gather(idx_flat, codebook.astype(jnp.float32))
    return out.reshape(*shape, codebook.shape[1])


# speedup vs baseline: 1.0019x; 1.0019x over previous
import functools

import jax
import jax.numpy as jnp
from jax import lax
from jax.experimental import pallas as pl
from jax.experimental.pallas import tpu as pltpu
from jax.experimental.pallas import tpu_sc as plsc

_NUM_CORES = 2
_NUM_SUBCORES = 16
_NW = _NUM_CORES * _NUM_SUBCORES
_CHUNK = 32
_GSUB = 1
_NBUF = 7


@functools.partial(jax.jit, static_argnames=())
def _sc_gather(idx_flat, codebook):
    B = idx_flat.shape[0]
    V, D = codebook.shape
    b_per_w = B // _NW
    n_chunks = b_per_w // _CHUNK
    C = _CHUNK
    mesh = plsc.VectorSubcoreMesh(core_axis_name="c", subcore_axis_name="s")

    nb = _NBUF
    ns = _GSUB
    S = C // ns
    row_bufs = [pltpu.VMEM((C, D), jnp.float32) for _ in range(nb)]
    g_sems = [pltpu.SemaphoreType.DMA for _ in range(nb)]
    w_sems = [pltpu.SemaphoreType.DMA for _ in range(nb)]

    @functools.partial(
        pl.kernel,
        out_type=jax.ShapeDtypeStruct((B, D), jnp.float32),
        mesh=mesh,
        scratch_types=[pltpu.VMEM((b_per_w,), jnp.int32)] + row_bufs + g_sems + w_sems,
    )
    def k(table_hbm, idx_hbm, out_hbm, idx_v, *bufs_and_sems):
        rows = bufs_and_sems[:nb]
        sg = bufs_and_sems[nb : 2 * nb]
        sw = bufs_and_sems[2 * nb : 3 * nb]
        wid = lax.axis_index("s") * _NUM_CORES + lax.axis_index("c")
        base = wid * b_per_w
        pltpu.sync_copy(idx_hbm.at[pl.ds(base, b_per_w)], idx_v)

        def gather(off, b):
            for h in range(ns):
                pltpu.async_copy(
                    table_hbm.at[idx_v.at[pl.ds(off + h * S, S)]],
                    rows[b].at[pl.ds(h * S, S)],
                    sg[b],
                )

        def wait_gather(b):
            for h in range(ns):
                pltpu.make_async_copy(
                    table_hbm.at[idx_v.at[pl.ds(0, S)]],
                    rows[b].at[pl.ds(0, S)],
                    sg[b],
                ).wait()

        def write(off, b):
            pltpu.async_copy(rows[b], out_hbm.at[pl.ds(base + off, C)], sw[b])

        def wait_write(b):
            pltpu.make_async_copy(rows[b], out_hbm.at[pl.ds(base, C)], sw[b]).wait()

        for b in range(nb):
            gather(b * C, b)

        n_main = (n_chunks // nb) - 1
        tail = n_chunks - (n_main + 1) * nb

        lag = 3
        @pl.loop(0, n_main)
        def _(g):
            off = g * nb * C
            for b in range(nb):
                wait_gather(b)
                write(off + b * C, b)
                bb = b - lag
                if bb >= 0:
                    wait_write(bb)
                    gather(off + (nb + bb) * C, bb)
            for bb in range(nb - lag, nb):
                wait_write(bb)
                gather(off + (nb + bb) * C, bb)

        last = n_main * nb * C
        for b in range(nb):
            wait_gather(b)
            write(last + b * C, b)
        for b in range(tail):
            wait_write(b)
            gather(last + (nb + b) * C, b)
        for b in range(tail):
            wait_gather(b)
            write(last + (nb + b) * C, b)
        for b in range(nb):
            wait_write(b)

    return k(codebook, idx_flat)


def kernel(indices, codebook):
    shape = indices.shape
    idx_flat = indices.reshape(-1).astype(jnp.int32)
    out = _sc_gather(idx_flat, codebook.astype(jnp.float32))
    return out.reshape(*shape, codebook.shape[1])
